# baseline (speedup 1.0000x reference)
import functools

import jax
import jax.numpy as jnp
from jax import lax
from jax.experimental import pallas as pl
from jax.experimental.pallas import tpu as pltpu
from jax.experimental.pallas import tpu_sc as plsc

N_NODES = 10000
N_PAD = 10240
N_EDGES = 160000
D_FEAT = 256
LANES = 16
BLANES = 2 * LANES
D_PACK = D_FEAT // 2

NUM_CORES = 2
NUM_SUBCORES = 16
NUM_WORKERS = NUM_CORES * NUM_SUBCORES
E_PAD = 163840
E_PER_W = E_PAD // NUM_WORKERS
CHUNK = 64
ROWS = 2 * CHUNK
GROUPS = CHUNK // LANES
N_CHUNKS = E_PER_W // CHUNK
IDX_PER_W = E_PER_W * 2
NBUF = 4
STAGE_ROWS = N_PAD // NUM_SUBCORES

_GATHER_DNUMS = lax.GatherDimensionNumbers(
    offset_dims=(), collapsed_slice_dims=(0,), start_index_map=(0,))


def _vshuffle(x, idx):
    return lax.gather(x, idx[:, None], _GATHER_DNUMS, slice_sizes=(1,),
                      mode=lax.GatherScatterMode.PROMISE_IN_BOUNDS)


def _edge_dot_body(h_hbm, comb_hbm, out_hbm,
                   idx_v, buf0, buf1, buf2, buf3, out_v,
                   sem0, sem1, sem2, sem3):
    cid = lax.axis_index("c")
    sid = lax.axis_index("s")
    wid = sid * NUM_CORES + cid
    base = pl.multiple_of(wid * E_PER_W, 8)

    pltpu.sync_copy(comb_hbm.at[pl.ds(pl.multiple_of(wid * IDX_PER_W, 8),
                                      IDX_PER_W)], idx_v)

    lane = lax.broadcasted_iota(jnp.int32, (LANES,), 0)
    perms = [lane ^ stride for stride in (8, 4, 2, 1)]

    def issue(j, buf, sem):
        off = pl.multiple_of(j * ROWS, 8)
        pltpu.async_copy(h_hbm.at[idx_v.at[pl.ds(off, ROWS)]], buf, sem)

    def compute(j, buf):
        def group_body(g, _):
            def edge_body(e, scores):
                row = g * LANES + e
                acc = jnp.zeros((LANES,), jnp.float32)
                himask = jnp.full((LANES,), -65536, jnp.int32)
                for d in range(D_PACK // LANES):
                    uw = buf[row, pl.ds(d * LANES, LANES)]
                    vw = buf[CHUNK + row, pl.ds(d * LANES, LANES)]
                    ua = lax.bitcast_convert_type(uw & himask, jnp.float32)
                    ub = lax.bitcast_convert_type(uw << 16, jnp.float32)
                    va = lax.bitcast_convert_type(vw & himask, jnp.float32)
                    vb = lax.bitcast_convert_type(vw << 16, jnp.float32)
                    acc = acc + ua * va
                    acc = acc + ub * vb
                for p in perms:
                    acc = acc + _vshuffle(acc, p)
                return jnp.where(lane == e, acc, scores)

            scores = lax.fori_loop(0, LANES, edge_body,
                                   jnp.zeros((LANES,), jnp.float32))
            out_v[pl.ds(j * CHUNK + g * LANES, LANES)] = scores
            return ()

        lax.fori_loop(0, GROUPS, group_body, ())

    bufs = (buf0, buf1, buf2, buf3)
    sems = (sem0, sem1, sem2, sem3)
    for k in range(NBUF - 1):
        issue(k, bufs[k], sems[k])

    def quad_body(iq, _):
        for b in range(NBUF):
            j = NBUF * iq + b
            ahead = (b + NBUF - 1) % NBUF

            @pl.when(j + NBUF - 1 < N_CHUNKS)
            def _():
                issue(j + NBUF - 1, bufs[ahead], sems[ahead])

            pltpu.make_async_copy(h_hbm.at[pl.ds(0, ROWS)], bufs[b],
                                  sems[b]).wait()
            compute(j, bufs[b])
        return ()

    lax.fori_loop(0, N_CHUNKS // NBUF, quad_body, ())
    pltpu.sync_copy(out_v, out_hbm.at[pl.ds(base, E_PER_W)])


@jax.jit
def _edge_dot(h_bf, comb):
    mesh = plsc.VectorSubcoreMesh(core_axis_name="c", subcore_axis_name="s")
    f = pl.kernel(
        _edge_dot_body,
        out_type=jax.ShapeDtypeStruct((E_PAD,), jnp.float32),
        mesh=mesh,
        scratch_types=[
            pltpu.VMEM((IDX_PER_W,), jnp.int32),
            pltpu.VMEM((ROWS, D_PACK), jnp.int32),
            pltpu.VMEM((ROWS, D_PACK), jnp.int32),
            pltpu.VMEM((ROWS, D_PACK), jnp.int32),
            pltpu.VMEM((ROWS, D_PACK), jnp.int32),
            pltpu.VMEM((E_PER_W,), jnp.float32),
            pltpu.SemaphoreType.DMA,
            pltpu.SemaphoreType.DMA,
            pltpu.SemaphoreType.DMA,
            pltpu.SemaphoreType.DMA,
        ],
    )
    return f(h_bf, comb)


def kernel(h, edge_index):
    h_bf = jnp.concatenate(
        [h.astype(jnp.bfloat16),
         jnp.zeros((N_PAD - N_NODES, D_FEAT), jnp.bfloat16)])
    h_pk = lax.bitcast_convert_type(
        h_bf.reshape(N_PAD, D_PACK, 2), jnp.int32)
    pad = E_PAD - N_EDGES
    src = jnp.concatenate([edge_index[0], jnp.zeros((pad,), jnp.int32)])
    dst = jnp.concatenate([edge_index[1], jnp.zeros((pad,), jnp.int32)])
    comb = jnp.concatenate(
        [src.reshape(-1, CHUNK), dst.reshape(-1, CHUNK)], axis=1).reshape(-1)
    score = _edge_dot(h_pk, comb)
    return score[:N_EDGES].reshape(N_EDGES, 1)


# speedup vs baseline: 2.3715x; 2.3715x over previous
import functools

import jax
import jax.numpy as jnp
from jax import lax
from jax.experimental import pallas as pl
from jax.experimental.pallas import tpu as pltpu
from jax.experimental.pallas import tpu_sc as plsc

N_NODES = 10000
N_PAD = 10240
N_EDGES = 160000
D_FEAT = 256
LANES = 16
BLANES = 2 * LANES
D_PACK = D_FEAT // 2

NUM_CORES = 2
NUM_SUBCORES = 16
NUM_WORKERS = NUM_CORES * NUM_SUBCORES
E_PAD = 163840
E_PER_W = E_PAD // NUM_WORKERS
CHUNK = 64
ROWS = 2 * CHUNK
GROUPS = CHUNK // LANES
N_CHUNKS = E_PER_W // CHUNK
IDX_PER_W = E_PER_W * 2
NBUF = 4
STAGE_ROWS = N_PAD // NUM_SUBCORES

_GATHER_DNUMS = lax.GatherDimensionNumbers(
    offset_dims=(), collapsed_slice_dims=(0,), start_index_map=(0,))


def _vshuffle(x, idx):
    return lax.gather(x, idx[:, None], _GATHER_DNUMS, slice_sizes=(1,),
                      mode=lax.GatherScatterMode.PROMISE_IN_BOUNDS)


def _edge_dot_body(h_hbm, comb_hbm, out_hbm,
                   h_sp, idx_v, buf0, buf1, buf2, buf3, out_v,
                   sem0, sem1, sem2, sem3):
    cid = lax.axis_index("c")
    sid = lax.axis_index("s")
    wid = sid * NUM_CORES + cid
    base = pl.multiple_of(wid * E_PER_W, 8)

    pltpu.sync_copy(comb_hbm.at[pl.ds(pl.multiple_of(wid * IDX_PER_W, 8),
                                      IDX_PER_W)], idx_v)
    srow = pl.multiple_of(sid * STAGE_ROWS, 8)
    pltpu.sync_copy(h_hbm.at[pl.ds(srow, STAGE_ROWS)],
                    h_sp.at[pl.ds(srow, STAGE_ROWS)])
    plsc.subcore_barrier()

    lane = lax.broadcasted_iota(jnp.int32, (LANES,), 0)
    perms = [lane ^ stride for stride in (8, 4, 2, 1)]

    def issue(j, buf, sem):
        off = pl.multiple_of(j * ROWS, 8)
        pltpu.sync_copy(h_sp.at[idx_v.at[pl.ds(off, ROWS)]], buf)

    def compute(j, buf):
        def group_body(g, _):
            def edge_body(e, scores):
                row = g * LANES + e
                acc = jnp.zeros((LANES,), jnp.float32)
                himask = jnp.full((LANES,), -65536, jnp.int32)
                for d in range(D_PACK // LANES):
                    uw = buf[row, pl.ds(d * LANES, LANES)]
                    vw = buf[CHUNK + row, pl.ds(d * LANES, LANES)]
                    ua = lax.bitcast_convert_type(uw & himask, jnp.float32)
                    ub = lax.bitcast_convert_type(uw << 16, jnp.float32)
                    va = lax.bitcast_convert_type(vw & himask, jnp.float32)
                    vb = lax.bitcast_convert_type(vw << 16, jnp.float32)
                    acc = acc + ua * va
                    acc = acc + ub * vb
                for p in perms:
                    acc = acc + _vshuffle(acc, p)
                return jnp.where(lane == e, acc, scores)

            scores = lax.fori_loop(0, LANES, edge_body,
                                   jnp.zeros((LANES,), jnp.float32))
            out_v[pl.ds(j * CHUNK + g * LANES, LANES)] = scores
            return ()

        lax.fori_loop(0, GROUPS, group_body, ())

    def chunk_body(j, _):
        issue(j, buf0, sem0)
        compute(j, buf0)
        return ()

    lax.fori_loop(0, N_CHUNKS, chunk_body, ())
    pltpu.sync_copy(out_v, out_hbm.at[pl.ds(base, E_PER_W)])


@jax.jit
def _edge_dot(h_bf, comb):
    mesh = plsc.VectorSubcoreMesh(core_axis_name="c", subcore_axis_name="s")
    f = pl.kernel(
        _edge_dot_body,
        out_type=jax.ShapeDtypeStruct((E_PAD,), jnp.float32),
        mesh=mesh,
        scratch_types=[
            pltpu.VMEM_SHARED((N_PAD, D_PACK), jnp.int32),
            pltpu.VMEM((IDX_PER_W,), jnp.int32),
            pltpu.VMEM((ROWS, D_PACK), jnp.int32),
            pltpu.VMEM((ROWS, D_PACK), jnp.int32),
            pltpu.VMEM((ROWS, D_PACK), jnp.int32),
            pltpu.VMEM((ROWS, D_PACK), jnp.int32),
            pltpu.VMEM((E_PER_W,), jnp.float32),
            pltpu.SemaphoreType.DMA,
            pltpu.SemaphoreType.DMA,
            pltpu.SemaphoreType.DMA,
            pltpu.SemaphoreType.DMA,
        ],
    )
    return f(h_bf, comb)


def kernel(h, edge_index):
    h_bf = jnp.concatenate(
        [h.astype(jnp.bfloat16),
         jnp.zeros((N_PAD - N_NODES, D_FEAT), jnp.bfloat16)])
    h_pk = lax.bitcast_convert_type(
        h_bf.reshape(N_PAD, D_PACK, 2), jnp.int32)
    pad = E_PAD - N_EDGES
    src = jnp.concatenate([edge_index[0], jnp.zeros((pad,), jnp.int32)])
    dst = jnp.concatenate([edge_index[1], jnp.zeros((pad,), jnp.int32)])
    comb = jnp.concatenate(
        [src.reshape(-1, CHUNK), dst.reshape(-1, CHUNK)], axis=1).reshape(-1)
    score = _edge_dot(h_pk, comb)
    return score[:N_EDGES].reshape(N_EDGES, 1)
